# Initial kernel scaffold; baseline (speedup 1.0000x reference)
import jax
import jax.numpy as jnp
from jax.experimental import pallas as pl


def kernel(x, edge_index, edge_attr, batch, emb, lin_W, lin_b, fW1, fb1, fW2, fb2, mW1, mb1, mW2, mb2, Wp1, bp1, Wp2, bp2):
    raise NotImplementedError("write your pallas kernel here")


# speedup vs baseline: 2.4977x; 2.4977x over previous
import jax
import jax.numpy as jnp
import numpy as np
from jax import lax
from jax.experimental import pallas as pl
from jax.experimental.pallas import tpu as pltpu
from jax.experimental.pallas import tpu_sc as plsc

H = 128
G = 50
NB = 3
N = 10000
E = 320000
NG = 512
LOG2 = float(np.log(2.0))
STEP = 30.0 / 49.0
COEFF = -0.5 / STEP ** 2

NBLK = 1000
EBLK = 3200
NC, NS = 2, 16
NW = NC * NS
EPW = E // NW
C = 80
NCHUNK = EPW // C
RPB = 624
TAIL = N - NS * RPB

_f32 = jnp.float32


def _ssp(v):
    return jnp.maximum(v, 0.0) + jnp.log(1.0 + jnp.exp(-jnp.abs(v))) - LOG2


def _embed_body(x_ref, emb_ref, o_ref):
    xb = x_ref[...]
    oh = (xb == lax.broadcasted_iota(jnp.int32, (NBLK, 10), 1)).astype(_f32)
    o_ref[...] = jnp.dot(oh, emb_ref[...], preferred_element_type=_f32)


_embed = pl.pallas_call(
    _embed_body,
    grid=(N // NBLK,),
    in_specs=[
        pl.BlockSpec((NBLK, 1), lambda i: (i, 0)),
        pl.BlockSpec((10, H), lambda i: (0, 0)),
    ],
    out_specs=pl.BlockSpec((NBLK, H), lambda i: (i, 0)),
    out_shape=jax.ShapeDtypeStruct((N, H), _f32),
)


def _filt_body(d_ref, fW1_ref, fb1_ref, fW2_ref, fb2_ref, o0, o1, o2):
    d = d_ref[...]
    offs = lax.broadcasted_iota(jnp.int32, (1, G), 1).astype(_f32) * STEP
    ea = jnp.exp(COEFF * (d - offs) ** 2)
    for b, o in enumerate((o0, o1, o2)):
        t = _ssp(jnp.dot(ea, fW1_ref[b], preferred_element_type=_f32) + fb1_ref[b])
        o[...] = jnp.dot(t, fW2_ref[b], preferred_element_type=_f32) + fb2_ref[b]


_filt = pl.pallas_call(
    _filt_body,
    grid=(E // EBLK,),
    in_specs=[
        pl.BlockSpec((EBLK, 1), lambda i: (i, 0)),
        pl.BlockSpec((NB, G, H), lambda i: (0, 0, 0)),
        pl.BlockSpec((NB, 1, H), lambda i: (0, 0, 0)),
        pl.BlockSpec((NB, H, H), lambda i: (0, 0, 0)),
        pl.BlockSpec((NB, 1, H), lambda i: (0, 0, 0)),
    ],
    out_specs=[pl.BlockSpec((EBLK, H), lambda i: (i, 0)) for _ in range(NB)],
    out_shape=[jax.ShapeDtypeStruct((E, H), _f32) for _ in range(NB)],
)


def _linear_body(nf_ref, W_ref, b_ref, o_ref):
    o_ref[...] = (
        jnp.dot(nf_ref[...], W_ref[...], preferred_element_type=_f32) + b_ref[...]
    )


_linear = pl.pallas_call(
    _linear_body,
    grid=(N // NBLK,),
    in_specs=[
        pl.BlockSpec((NBLK, H), lambda i: (i, 0)),
        pl.BlockSpec((H, H), lambda i: (0, 0)),
        pl.BlockSpec((1, H), lambda i: (0, 0)),
    ],
    out_specs=pl.BlockSpec((NBLK, H), lambda i: (i, 0)),
    out_shape=jax.ShapeDtypeStruct((N, H), _f32),
)


def _update_body(aggp_ref, nf_ref, mW1_ref, mb1_ref, mW2_ref, mb2_ref, o_ref):
    agg = aggp_ref[0] + aggp_ref[1]
    t = _ssp(jnp.dot(agg, mW1_ref[...], preferred_element_type=_f32) + mb1_ref[...])
    o_ref[...] = (
        nf_ref[...]
        + jnp.dot(t, mW2_ref[...], preferred_element_type=_f32)
        + mb2_ref[...]
    )


_update = pl.pallas_call(
    _update_body,
    grid=(N // NBLK,),
    in_specs=[
        pl.BlockSpec((NC, NBLK, H), lambda i: (0, i, 0)),
        pl.BlockSpec((NBLK, H), lambda i: (i, 0)),
        pl.BlockSpec((H, H), lambda i: (0, 0)),
        pl.BlockSpec((1, H), lambda i: (0, 0)),
        pl.BlockSpec((H, H), lambda i: (0, 0)),
        pl.BlockSpec((1, H), lambda i: (0, 0)),
    ],
    out_specs=pl.BlockSpec((NBLK, H), lambda i: (i, 0)),
    out_shape=jax.ShapeDtypeStruct((N, H), _f32),
)


def _final_body(nf_ref, batch_ref, Wp1_ref, bp1_ref, Wp2_ref, bp2_ref, o_ref):
    i = pl.program_id(0)
    t = _ssp(jnp.dot(nf_ref[...], Wp1_ref[...], preferred_element_type=_f32) + bp1_ref[...])
    site = jnp.dot(t, Wp2_ref[...], preferred_element_type=_f32) + bp2_ref[...]
    g = lax.broadcasted_iota(jnp.int32, (NG, NBLK), 0)
    mask = (batch_ref[0] == g).astype(_f32)
    contrib = jnp.dot(mask, site, preferred_element_type=_f32)

    @pl.when(i == 0)
    def _():
        o_ref[...] = jnp.zeros_like(o_ref)

    o_ref[...] += contrib


_final = pl.pallas_call(
    _final_body,
    grid=(N // NBLK,),
    in_specs=[
        pl.BlockSpec((NBLK, H), lambda i: (i, 0)),
        pl.BlockSpec((1, 1, NBLK), lambda i: (i, 0, 0)),
        pl.BlockSpec((H, H // 2), lambda i: (0, 0)),
        pl.BlockSpec((1, H // 2), lambda i: (0, 0)),
        pl.BlockSpec((H // 2, 1), lambda i: (0, 0)),
        pl.BlockSpec((1, 1), lambda i: (0, 0)),
    ],
    out_specs=pl.BlockSpec((NG, 1), lambda i: (0, 0)),
    out_shape=jax.ShapeDtypeStruct((NG, 1), _f32),
)


def _conv_body(h1_hbm, filt_hbm, src_hbm, dst_hbm, zeros_hbm, out_hbm,
               idx_src, idx_dst, rows, filt_v, agg_sh, sem):
    c = lax.axis_index("c")
    s = lax.axis_index("s")
    wid = s * NC + c
    rb = pl.multiple_of(s * RPB, 8)
    pltpu.sync_copy(zeros_hbm.at[pl.ds(rb, RPB)], agg_sh.at[pl.ds(rb, RPB)])

    @pl.when(s == NS - 1)
    def _():
        pltpu.sync_copy(zeros_hbm.at[pl.ds(NS * RPB, TAIL)],
                        agg_sh.at[pl.ds(NS * RPB, TAIL)])

    plsc.subcore_barrier()

    base = wid * EPW

    def chunk(k, carry):
        eb = pl.multiple_of(base + k * C, 8)
        pltpu.sync_copy(src_hbm.at[pl.ds(eb, C)], idx_src)
        pltpu.sync_copy(dst_hbm.at[pl.ds(eb, C)], idx_dst)
        pltpu.async_copy(h1_hbm.at[idx_src], rows, sem).wait()
        pltpu.sync_copy(filt_hbm.at[pl.ds(eb, C)], filt_v)

        def mul(e, carry2):
            for j in range(H // 16):
                sl = pl.ds(j * 16, 16)
                rows[e, sl] = rows[e, sl] * filt_v[e, sl]
            return carry2

        lax.fori_loop(0, C, mul, 0)
        pltpu.sync_copy(rows, agg_sh.at[idx_dst], add=True)
        return carry

    lax.fori_loop(0, NCHUNK, chunk, 0)
    plsc.subcore_barrier()
    pltpu.sync_copy(agg_sh.at[pl.ds(rb, RPB)], out_hbm.at[c, pl.ds(rb, RPB)])

    @pl.when(s == NS - 1)
    def _():
        pltpu.sync_copy(agg_sh.at[pl.ds(NS * RPB, TAIL)],
                        out_hbm.at[c, pl.ds(NS * RPB, TAIL)])


_conv = pl.kernel(
    _conv_body,
    out_type=jax.ShapeDtypeStruct((NC, N, H), _f32),
    mesh=plsc.VectorSubcoreMesh(
        core_axis_name="c", subcore_axis_name="s", num_cores=NC, num_subcores=NS
    ),
    scratch_types=[
        pltpu.VMEM((C,), jnp.int32),
        pltpu.VMEM((C,), jnp.int32),
        pltpu.VMEM((C, H), _f32),
        pltpu.VMEM((C, H), _f32),
        pltpu.VMEM_SHARED((N, H), _f32),
        pltpu.SemaphoreType.DMA,
    ],
)


def kernel(x, edge_index, edge_attr, batch, emb, lin_W, lin_b,
           fW1, fb1, fW2, fb2, mW1, mb1, mW2, mb2, Wp1, bp1, Wp2, bp2):
    src = edge_index[0]
    dst = edge_index[1]
    x2 = x.reshape(N, 1).astype(jnp.int32)
    d2 = edge_attr.reshape(E, 1)
    batch2 = batch.reshape(N // NBLK, 1, NBLK).astype(jnp.int32)
    zeros = jnp.zeros((N, H), _f32)

    nf = _embed(x2, emb)
    filts = _filt(d2, fW1, fb1.reshape(NB, 1, H), fW2, fb2.reshape(NB, 1, H))
    for b in range(NB):
        h1 = _linear(nf, lin_W[b], lin_b[b].reshape(1, H))
        aggp = _conv(h1, filts[b], src, dst, zeros)
        nf = _update(aggp, nf, mW1[b], mb1[b].reshape(1, H),
                     mW2[b], mb2[b].reshape(1, H))
    out2 = _final(nf, batch2, Wp1, bp1.reshape(1, H // 2), Wp2, bp2.reshape(1, 1))
    return out2.reshape(NG)


# speedup vs baseline: 3.8267x; 1.5321x over previous
import jax
import jax.numpy as jnp
import numpy as np
from jax import lax
from jax.experimental import pallas as pl
from jax.experimental.pallas import tpu as pltpu
from jax.experimental.pallas import tpu_sc as plsc

H = 128
G = 50
NB = 3
N = 10000
E = 320000
NG = 512
LOG2 = float(np.log(2.0))
STEP = 30.0 / 49.0
COEFF = -0.5 / STEP ** 2

NBLK = 1000
EBLK = 3200
NC, NS = 2, 16
NW = NC * NS
EPW = E // NW
C = 40
NCHUNK = EPW // C
RPB = 624
TAIL = N - NS * RPB

_f32 = jnp.float32


def _ssp(v):
    return jnp.maximum(v, 0.0) + jnp.log(1.0 + jnp.exp(-jnp.abs(v))) - LOG2


def _embed_body(x_ref, emb_ref, o_ref):
    xb = x_ref[...]
    oh = (xb == lax.broadcasted_iota(jnp.int32, (NBLK, 10), 1)).astype(_f32)
    o_ref[...] = jnp.dot(oh, emb_ref[...], preferred_element_type=_f32)


_embed = pl.pallas_call(
    _embed_body,
    grid=(N // NBLK,),
    in_specs=[
        pl.BlockSpec((NBLK, 1), lambda i: (i, 0)),
        pl.BlockSpec((10, H), lambda i: (0, 0)),
    ],
    out_specs=pl.BlockSpec((NBLK, H), lambda i: (i, 0)),
    out_shape=jax.ShapeDtypeStruct((N, H), _f32),
)


def _filt_body(d_ref, fW1_ref, fb1_ref, fW2_ref, fb2_ref, o0, o1, o2):
    d = d_ref[...]
    offs = lax.broadcasted_iota(jnp.int32, (1, G), 1).astype(_f32) * STEP
    ea = jnp.exp(COEFF * (d - offs) ** 2)
    for b, o in enumerate((o0, o1, o2)):
        t = _ssp(jnp.dot(ea, fW1_ref[b], preferred_element_type=_f32) + fb1_ref[b])
        o[...] = jnp.dot(t, fW2_ref[b], preferred_element_type=_f32) + fb2_ref[b]


_filt = pl.pallas_call(
    _filt_body,
    grid=(E // EBLK,),
    in_specs=[
        pl.BlockSpec((EBLK, 1), lambda i: (i, 0)),
        pl.BlockSpec((NB, G, H), lambda i: (0, 0, 0)),
        pl.BlockSpec((NB, 1, H), lambda i: (0, 0, 0)),
        pl.BlockSpec((NB, H, H), lambda i: (0, 0, 0)),
        pl.BlockSpec((NB, 1, H), lambda i: (0, 0, 0)),
    ],
    out_specs=[pl.BlockSpec((EBLK, H), lambda i: (i, 0)) for _ in range(NB)],
    out_shape=[jax.ShapeDtypeStruct((E, H), _f32) for _ in range(NB)],
)


def _linear_body(nf_ref, W_ref, b_ref, o_ref):
    o_ref[...] = (
        jnp.dot(nf_ref[...], W_ref[...], preferred_element_type=_f32) + b_ref[...]
    )


_linear = pl.pallas_call(
    _linear_body,
    grid=(N // NBLK,),
    in_specs=[
        pl.BlockSpec((NBLK, H), lambda i: (i, 0)),
        pl.BlockSpec((H, H), lambda i: (0, 0)),
        pl.BlockSpec((1, H), lambda i: (0, 0)),
    ],
    out_specs=pl.BlockSpec((NBLK, H), lambda i: (i, 0)),
    out_shape=jax.ShapeDtypeStruct((N, H), _f32),
)


def _update_body(aggp_ref, nf_ref, mW1_ref, mb1_ref, mW2_ref, mb2_ref, o_ref):
    agg = aggp_ref[0] + aggp_ref[1]
    t = _ssp(jnp.dot(agg, mW1_ref[...], preferred_element_type=_f32) + mb1_ref[...])
    o_ref[...] = (
        nf_ref[...]
        + jnp.dot(t, mW2_ref[...], preferred_element_type=_f32)
        + mb2_ref[...]
    )


_update = pl.pallas_call(
    _update_body,
    grid=(N // NBLK,),
    in_specs=[
        pl.BlockSpec((NC, NBLK, H), lambda i: (0, i, 0)),
        pl.BlockSpec((NBLK, H), lambda i: (i, 0)),
        pl.BlockSpec((H, H), lambda i: (0, 0)),
        pl.BlockSpec((1, H), lambda i: (0, 0)),
        pl.BlockSpec((H, H), lambda i: (0, 0)),
        pl.BlockSpec((1, H), lambda i: (0, 0)),
    ],
    out_specs=pl.BlockSpec((NBLK, H), lambda i: (i, 0)),
    out_shape=jax.ShapeDtypeStruct((N, H), _f32),
)


def _final_body(nf_ref, batch_ref, Wp1_ref, bp1_ref, Wp2_ref, bp2_ref, o_ref):
    i = pl.program_id(0)
    t = _ssp(jnp.dot(nf_ref[...], Wp1_ref[...], preferred_element_type=_f32) + bp1_ref[...])
    site = jnp.dot(t, Wp2_ref[...], preferred_element_type=_f32) + bp2_ref[...]
    g = lax.broadcasted_iota(jnp.int32, (NG, NBLK), 0)
    mask = (batch_ref[0] == g).astype(_f32)
    contrib = jnp.dot(mask, site, preferred_element_type=_f32)

    @pl.when(i == 0)
    def _():
        o_ref[...] = jnp.zeros_like(o_ref)

    o_ref[...] += contrib


_final = pl.pallas_call(
    _final_body,
    grid=(N // NBLK,),
    in_specs=[
        pl.BlockSpec((NBLK, H), lambda i: (i, 0)),
        pl.BlockSpec((1, 1, NBLK), lambda i: (i, 0, 0)),
        pl.BlockSpec((H, H // 2), lambda i: (0, 0)),
        pl.BlockSpec((1, H // 2), lambda i: (0, 0)),
        pl.BlockSpec((H // 2, 1), lambda i: (0, 0)),
        pl.BlockSpec((1, 1), lambda i: (0, 0)),
    ],
    out_specs=pl.BlockSpec((NG, 1), lambda i: (0, 0)),
    out_shape=jax.ShapeDtypeStruct((NG, 1), _f32),
)


def _conv_body(h1_hbm, filt_hbm, src_hbm, dst_hbm, zeros_hbm, out_hbm,
               idx_src, idx_dst, rows, filt_v, agg_sh,
               gsem, fsem, dsem, ssem, isem):
    c = lax.axis_index("c")
    s = lax.axis_index("s")
    wid = s * NC + c
    base = pl.multiple_of(wid * EPW, 8)
    rb = pl.multiple_of(s * RPB, 8)
    pltpu.sync_copy(zeros_hbm.at[pl.ds(rb, RPB)], agg_sh.at[pl.ds(rb, RPB)])

    @pl.when(s == NS - 1)
    def _():
        pltpu.sync_copy(zeros_hbm.at[pl.ds(NS * RPB, TAIL)],
                        agg_sh.at[pl.ds(NS * RPB, TAIL)])

    plsc.subcore_barrier()

    def issue(k, p, wait_prev):
        if wait_prev:
            pltpu.make_async_copy(h1_hbm.at[pl.ds(0, C)], rows[p], ssem[p]).wait()
        eb = pl.multiple_of(base + k * C, 8)
        pltpu.async_copy(src_hbm.at[pl.ds(eb, C)], idx_src[p], isem[p])
        pltpu.async_copy(dst_hbm.at[pl.ds(eb, C)], idx_dst[p], dsem[p])
        pltpu.make_async_copy(src_hbm.at[pl.ds(0, C)], idx_src[p], isem[p]).wait()
        pltpu.async_copy(h1_hbm.at[idx_src[p]], rows[p], gsem[p])
        pltpu.async_copy(filt_hbm.at[pl.ds(eb, C)], filt_v[p], fsem[p])

    def process(p):
        pltpu.make_async_copy(h1_hbm.at[pl.ds(0, C)], rows[p], gsem[p]).wait()
        pltpu.make_async_copy(filt_hbm.at[pl.ds(0, C)], filt_v[p], fsem[p]).wait()
        pltpu.make_async_copy(dst_hbm.at[pl.ds(0, C)], idx_dst[p], dsem[p]).wait()

        @plsc.parallel_loop(0, C, step=1, unroll=2)
        def _(e):
            for j in range(H // 16):
                sl = pl.ds(j * 16, 16)
                rows[p][e, sl] = rows[p][e, sl] * filt_v[p][e, sl]

        pltpu.async_copy(rows[p], agg_sh.at[idx_dst[p]], ssem[p], add=True)

    issue(0, 0, False)
    issue(1, 1, False)
    process(0)
    issue(2, 2, False)

    def triple(k3, carry):
        a = 1 + 3 * k3
        for j in range(3):
            k = a + j

            @pl.when(k + 2 < NCHUNK)
            def _(k=k, j=j):
                issue(k + 2, j, True)

            process((1 + j) % 3)
        return carry

    lax.fori_loop(0, (NCHUNK - 1) // 3, triple, 0)
    for p in range(3):
        pltpu.make_async_copy(h1_hbm.at[pl.ds(0, C)], rows[p], ssem[p]).wait()
    plsc.subcore_barrier()
    pltpu.sync_copy(agg_sh.at[pl.ds(rb, RPB)], out_hbm.at[c, pl.ds(rb, RPB)])

    @pl.when(s == NS - 1)
    def _():
        pltpu.sync_copy(agg_sh.at[pl.ds(NS * RPB, TAIL)],
                        out_hbm.at[c, pl.ds(NS * RPB, TAIL)])


_conv = pl.kernel(
    _conv_body,
    out_type=jax.ShapeDtypeStruct((NC, N, H), _f32),
    mesh=plsc.VectorSubcoreMesh(
        core_axis_name="c", subcore_axis_name="s", num_cores=NC, num_subcores=NS
    ),
    scratch_types=[
        [pltpu.VMEM((C,), jnp.int32) for _ in range(3)],
        [pltpu.VMEM((C,), jnp.int32) for _ in range(3)],
        [pltpu.VMEM((C, H), _f32) for _ in range(3)],
        [pltpu.VMEM((C, H), _f32) for _ in range(3)],
        pltpu.VMEM_SHARED((N, H), _f32),
        [pltpu.SemaphoreType.DMA for _ in range(3)],
        [pltpu.SemaphoreType.DMA for _ in range(3)],
        [pltpu.SemaphoreType.DMA for _ in range(3)],
        [pltpu.SemaphoreType.DMA for _ in range(3)],
        [pltpu.SemaphoreType.DMA for _ in range(3)],
    ],
)


def kernel(x, edge_index, edge_attr, batch, emb, lin_W, lin_b,
           fW1, fb1, fW2, fb2, mW1, mb1, mW2, mb2, Wp1, bp1, Wp2, bp2):
    src = edge_index[0]
    dst = edge_index[1]
    x2 = x.reshape(N, 1).astype(jnp.int32)
    d2 = edge_attr.reshape(E, 1)
    batch2 = batch.reshape(N // NBLK, 1, NBLK).astype(jnp.int32)
    zeros = jnp.zeros((N, H), _f32)

    nf = _embed(x2, emb)
    filts = _filt(d2, fW1, fb1.reshape(NB, 1, H), fW2, fb2.reshape(NB, 1, H))
    for b in range(NB):
        h1 = _linear(nf, lin_W[b], lin_b[b].reshape(1, H))
        aggp = _conv(h1, filts[b], src, dst, zeros)
        nf = _update(aggp, nf, mW1[b], mb1[b].reshape(1, H),
                     mW2[b], mb2[b].reshape(1, H))
    out2 = _final(nf, batch2, Wp1, bp1.reshape(1, H // 2), Wp2, bp2.reshape(1, 1))
    return out2.reshape(NG)


# speedup vs baseline: 4.0572x; 1.0602x over previous
import jax
import jax.numpy as jnp
import numpy as np
from jax import lax
from jax.experimental import pallas as pl
from jax.experimental.pallas import tpu as pltpu
from jax.experimental.pallas import tpu_sc as plsc

H = 128
G = 50
NB = 3
N = 10000
E = 320000
NG = 512
LOG2 = float(np.log(2.0))
STEP = 30.0 / 49.0
COEFF = -0.5 / STEP ** 2

NBLK = 1000
EBLK = 3200
NC, NS = 2, 16
NW = NC * NS
EPW = E // NW
C = 40
NCHUNK = EPW // C
RPB = 624
TAIL = N - NS * RPB

_f32 = jnp.float32


def _ssp(v):
    return jnp.maximum(v, 0.0) + jnp.log(1.0 + jnp.exp(-jnp.abs(v))) - LOG2


def _embed_body(x_ref, emb_ref, o_ref):
    xb = x_ref[...]
    oh = (xb == lax.broadcasted_iota(jnp.int32, (NBLK, 10), 1)).astype(_f32)
    o_ref[...] = jnp.dot(oh, emb_ref[...], preferred_element_type=_f32)


_embed = pl.pallas_call(
    _embed_body,
    grid=(N // NBLK,),
    in_specs=[
        pl.BlockSpec((NBLK, 1), lambda i: (i, 0)),
        pl.BlockSpec((10, H), lambda i: (0, 0)),
    ],
    out_specs=pl.BlockSpec((NBLK, H), lambda i: (i, 0)),
    out_shape=jax.ShapeDtypeStruct((N, H), _f32),
)


def _filt_body(d_ref, fW1_ref, fb1_ref, fW2_ref, fb2_ref, o_ref):
    d = d_ref[...]
    offs = lax.broadcasted_iota(jnp.int32, (1, G), 1).astype(_f32) * STEP
    ea = jnp.exp(COEFF * (d - offs) ** 2)
    t = _ssp(jnp.dot(ea, fW1_ref[...], preferred_element_type=_f32) + fb1_ref[...])
    o_ref[...] = jnp.dot(t, fW2_ref[...], preferred_element_type=_f32) + fb2_ref[...]


_filt = pl.pallas_call(
    _filt_body,
    grid=(E // EBLK,),
    in_specs=[
        pl.BlockSpec((EBLK, 1), lambda i: (i, 0)),
        pl.BlockSpec((G, H), lambda i: (0, 0)),
        pl.BlockSpec((1, H), lambda i: (0, 0)),
        pl.BlockSpec((H, H), lambda i: (0, 0)),
        pl.BlockSpec((1, H), lambda i: (0, 0)),
    ],
    out_specs=pl.BlockSpec((EBLK, H), lambda i: (i, 0)),
    out_shape=jax.ShapeDtypeStruct((E, H), _f32),
)


def _linear_body(nf_ref, W_ref, b_ref, o_ref):
    o_ref[...] = (
        jnp.dot(nf_ref[...], W_ref[...], preferred_element_type=_f32) + b_ref[...]
    )


_linear = pl.pallas_call(
    _linear_body,
    grid=(N // NBLK,),
    in_specs=[
        pl.BlockSpec((NBLK, H), lambda i: (i, 0)),
        pl.BlockSpec((H, H), lambda i: (0, 0)),
        pl.BlockSpec((1, H), lambda i: (0, 0)),
    ],
    out_specs=pl.BlockSpec((NBLK, H), lambda i: (i, 0)),
    out_shape=jax.ShapeDtypeStruct((N, H), _f32),
)


def _update_body(aggp_ref, nf_ref, mW1_ref, mb1_ref, mW2_ref, mb2_ref,
                 lW_ref, lb_ref, o_ref, h1_ref):
    agg = aggp_ref[0] + aggp_ref[1]
    t = _ssp(jnp.dot(agg, mW1_ref[...], preferred_element_type=_f32) + mb1_ref[...])
    nf_new = (
        nf_ref[...]
        + jnp.dot(t, mW2_ref[...], preferred_element_type=_f32)
        + mb2_ref[...]
    )
    o_ref[...] = nf_new
    h1_ref[...] = jnp.dot(nf_new, lW_ref[...], preferred_element_type=_f32) + lb_ref[...]


_update = pl.pallas_call(
    _update_body,
    grid=(N // NBLK,),
    in_specs=[
        pl.BlockSpec((NC, NBLK, H), lambda i: (0, i, 0)),
        pl.BlockSpec((NBLK, H), lambda i: (i, 0)),
        pl.BlockSpec((H, H), lambda i: (0, 0)),
        pl.BlockSpec((1, H), lambda i: (0, 0)),
        pl.BlockSpec((H, H), lambda i: (0, 0)),
        pl.BlockSpec((1, H), lambda i: (0, 0)),
        pl.BlockSpec((H, H), lambda i: (0, 0)),
        pl.BlockSpec((1, H), lambda i: (0, 0)),
    ],
    out_specs=[pl.BlockSpec((NBLK, H), lambda i: (i, 0)) for _ in range(2)],
    out_shape=[jax.ShapeDtypeStruct((N, H), _f32) for _ in range(2)],
)


def _final_body(nf_ref, batch_ref, Wp1_ref, bp1_ref, Wp2_ref, bp2_ref, o_ref):
    i = pl.program_id(0)
    t = _ssp(jnp.dot(nf_ref[...], Wp1_ref[...], preferred_element_type=_f32) + bp1_ref[...])
    site = jnp.dot(t, Wp2_ref[...], preferred_element_type=_f32) + bp2_ref[...]
    g = lax.broadcasted_iota(jnp.int32, (NG, NBLK), 0)
    mask = (batch_ref[0] == g).astype(_f32)
    contrib = jnp.dot(mask, site, preferred_element_type=_f32)

    @pl.when(i == 0)
    def _():
        o_ref[...] = jnp.zeros_like(o_ref)

    o_ref[...] += contrib


_final = pl.pallas_call(
    _final_body,
    grid=(N // NBLK,),
    in_specs=[
        pl.BlockSpec((NBLK, H), lambda i: (i, 0)),
        pl.BlockSpec((1, 1, NBLK), lambda i: (i, 0, 0)),
        pl.BlockSpec((H, H // 2), lambda i: (0, 0)),
        pl.BlockSpec((1, H // 2), lambda i: (0, 0)),
        pl.BlockSpec((H // 2, 1), lambda i: (0, 0)),
        pl.BlockSpec((1, 1), lambda i: (0, 0)),
    ],
    out_specs=pl.BlockSpec((NG, 1), lambda i: (0, 0)),
    out_shape=jax.ShapeDtypeStruct((NG, 1), _f32),
)


def _conv_body(h1_hbm, filt_hbm, src_hbm, dst_hbm, zeros_hbm, out_hbm,
               idx_src, idx_dst, rows, filt_v, agg_sh,
               gsem, fsem, dsem, ssem, isem):
    c = lax.axis_index("c")
    s = lax.axis_index("s")
    wid = s * NC + c
    base = pl.multiple_of(wid * EPW, 8)
    rb = pl.multiple_of(s * RPB, 8)
    pltpu.sync_copy(zeros_hbm.at[pl.ds(rb, RPB)], agg_sh.at[pl.ds(rb, RPB)])

    @pl.when(s == NS - 1)
    def _():
        pltpu.sync_copy(zeros_hbm.at[pl.ds(NS * RPB, TAIL)],
                        agg_sh.at[pl.ds(NS * RPB, TAIL)])

    plsc.subcore_barrier()

    def issue(k, p, wait_prev):
        if wait_prev:
            pltpu.make_async_copy(h1_hbm.at[pl.ds(0, C)], rows[p], ssem[p]).wait()
        eb = pl.multiple_of(base + k * C, 8)
        pltpu.async_copy(src_hbm.at[pl.ds(eb, C)], idx_src[p], isem[p])
        pltpu.async_copy(dst_hbm.at[pl.ds(eb, C)], idx_dst[p], dsem[p])
        pltpu.make_async_copy(src_hbm.at[pl.ds(0, C)], idx_src[p], isem[p]).wait()
        pltpu.async_copy(h1_hbm.at[idx_src[p]], rows[p], gsem[p])
        pltpu.async_copy(filt_hbm.at[pl.ds(eb, C)], filt_v[p], fsem[p])

    def process(p):
        pltpu.make_async_copy(h1_hbm.at[pl.ds(0, C)], rows[p], gsem[p]).wait()
        pltpu.make_async_copy(filt_hbm.at[pl.ds(0, C)], filt_v[p], fsem[p]).wait()
        pltpu.make_async_copy(dst_hbm.at[pl.ds(0, C)], idx_dst[p], dsem[p]).wait()

        @plsc.parallel_loop(0, C, step=1, unroll=2)
        def _(e):
            for j in range(H // 16):
                sl = pl.ds(j * 16, 16)
                rows[p][e, sl] = rows[p][e, sl] * filt_v[p][e, sl]

        pltpu.async_copy(rows[p], agg_sh.at[idx_dst[p]], ssem[p], add=True)

    issue(0, 0, False)
    issue(1, 1, False)
    process(0)
    issue(2, 2, False)

    def triple(k3, carry):
        a = 1 + 3 * k3
        for j in range(3):
            k = a + j

            @pl.when(k + 2 < NCHUNK)
            def _(k=k, j=j):
                issue(k + 2, j, True)

            process((1 + j) % 3)
        return carry

    lax.fori_loop(0, (NCHUNK - 1) // 3, triple, 0)
    for p in range(3):
        pltpu.make_async_copy(h1_hbm.at[pl.ds(0, C)], rows[p], ssem[p]).wait()
    plsc.subcore_barrier()
    pltpu.sync_copy(agg_sh.at[pl.ds(rb, RPB)], out_hbm.at[c, pl.ds(rb, RPB)])

    @pl.when(s == NS - 1)
    def _():
        pltpu.sync_copy(agg_sh.at[pl.ds(NS * RPB, TAIL)],
                        out_hbm.at[c, pl.ds(NS * RPB, TAIL)])


_conv = pl.kernel(
    _conv_body,
    out_type=jax.ShapeDtypeStruct((NC, N, H), _f32),
    mesh=plsc.VectorSubcoreMesh(
        core_axis_name="c", subcore_axis_name="s", num_cores=NC, num_subcores=NS
    ),
    scratch_types=[
        [pltpu.VMEM((C,), jnp.int32) for _ in range(3)],
        [pltpu.VMEM((C,), jnp.int32) for _ in range(3)],
        [pltpu.VMEM((C, H), _f32) for _ in range(3)],
        [pltpu.VMEM((C, H), _f32) for _ in range(3)],
        pltpu.VMEM_SHARED((N, H), _f32),
        [pltpu.SemaphoreType.DMA for _ in range(3)],
        [pltpu.SemaphoreType.DMA for _ in range(3)],
        [pltpu.SemaphoreType.DMA for _ in range(3)],
        [pltpu.SemaphoreType.DMA for _ in range(3)],
        [pltpu.SemaphoreType.DMA for _ in range(3)],
    ],
)


def kernel(x, edge_index, edge_attr, batch, emb, lin_W, lin_b,
           fW1, fb1, fW2, fb2, mW1, mb1, mW2, mb2, Wp1, bp1, Wp2, bp2):
    src = edge_index[0]
    dst = edge_index[1]
    x2 = x.reshape(N, 1).astype(jnp.int32)
    d2 = edge_attr.reshape(E, 1)
    batch2 = batch.reshape(N // NBLK, 1, NBLK).astype(jnp.int32)
    zeros = jnp.zeros((N, H), _f32)

    nf = _embed(x2, emb)
    h1 = _linear(nf, lin_W[0], lin_b[0].reshape(1, H))
    for b in range(NB):
        filt_b = _filt(d2, fW1[b], fb1[b].reshape(1, H), fW2[b], fb2[b].reshape(1, H))
        aggp = _conv(h1, filt_b, src, dst, zeros)
        nxt = (b + 1) % NB
        nf, h1 = _update(aggp, nf, mW1[b], mb1[b].reshape(1, H),
                         mW2[b], mb2[b].reshape(1, H),
                         lin_W[nxt], lin_b[nxt].reshape(1, H))
    out2 = _final(nf, batch2, Wp1, bp1.reshape(1, H // 2), Wp2, bp2.reshape(1, 1))
    return out2.reshape(NG)


# speedup vs baseline: 4.9861x; 1.2290x over previous
import jax
import jax.numpy as jnp
import numpy as np
from jax import lax
from jax.experimental import pallas as pl
from jax.experimental.pallas import tpu as pltpu
from jax.experimental.pallas import tpu_sc as plsc

H = 128
G = 50
NB = 3
N = 10000
E = 320000
NG = 512
LOG2 = float(np.log(2.0))
STEP = 30.0 / 49.0
COEFF = -0.5 / STEP ** 2

NBLK = 1000
EBLK = 3200
NC, NS = 2, 16
NW = NC * NS
EPW = E // NW
C = 40
NCHUNK = EPW // C
RPB = 624
TAIL = N - NS * RPB

_f32 = jnp.float32


def _ssp(v):
    return jnp.maximum(v, 0.0) + jnp.log(1.0 + jnp.exp(-jnp.abs(v))) - LOG2


def _embed_body(x_ref, emb_ref, o_ref):
    xb = x_ref[...]
    oh = (xb == lax.broadcasted_iota(jnp.int32, (NBLK, 10), 1)).astype(_f32)
    o_ref[...] = jnp.dot(oh, emb_ref[...], preferred_element_type=_f32, precision=lax.Precision.HIGHEST)


_embed = pl.pallas_call(
    _embed_body,
    grid=(N // NBLK,),
    in_specs=[
        pl.BlockSpec((NBLK, 1), lambda i: (i, 0)),
        pl.BlockSpec((10, H), lambda i: (0, 0)),
    ],
    out_specs=pl.BlockSpec((NBLK, H), lambda i: (i, 0)),
    out_shape=jax.ShapeDtypeStruct((N, H), _f32),
)


def _filt_body(d_ref, fW1_ref, fb1_ref, fW2_ref, fb2_ref, o_ref):
    d = d_ref[...]
    offs = lax.broadcasted_iota(jnp.int32, (1, G), 1).astype(_f32) * STEP
    ea = jnp.exp(COEFF * (d - offs) ** 2)
    t = _ssp(jnp.dot(ea, fW1_ref[...], preferred_element_type=_f32) + fb1_ref[...])
    o_ref[...] = jnp.dot(t, fW2_ref[...], preferred_element_type=_f32) + fb2_ref[...]


_filt = pl.pallas_call(
    _filt_body,
    grid=(E // EBLK,),
    in_specs=[
        pl.BlockSpec((EBLK, 1), lambda i: (i, 0)),
        pl.BlockSpec((G, H), lambda i: (0, 0)),
        pl.BlockSpec((1, H), lambda i: (0, 0)),
        pl.BlockSpec((H, H), lambda i: (0, 0)),
        pl.BlockSpec((1, H), lambda i: (0, 0)),
    ],
    out_specs=pl.BlockSpec((EBLK, H), lambda i: (i, 0)),
    out_shape=jax.ShapeDtypeStruct((E, H), _f32),
)


def _linear_body(nf_ref, W_ref, b_ref, o_ref):
    o_ref[...] = (
        jnp.dot(nf_ref[...], W_ref[...], preferred_element_type=_f32) + b_ref[...]
    )


_linear = pl.pallas_call(
    _linear_body,
    grid=(N // NBLK,),
    in_specs=[
        pl.BlockSpec((NBLK, H), lambda i: (i, 0)),
        pl.BlockSpec((H, H), lambda i: (0, 0)),
        pl.BlockSpec((1, H), lambda i: (0, 0)),
    ],
    out_specs=pl.BlockSpec((NBLK, H), lambda i: (i, 0)),
    out_shape=jax.ShapeDtypeStruct((N, H), _f32),
)


def _update_body(aggp_ref, nf_ref, mW1_ref, mb1_ref, mW2_ref, mb2_ref,
                 lW_ref, lb_ref, o_ref, h1_ref):
    agg = aggp_ref[0] + aggp_ref[1]
    t = _ssp(jnp.dot(agg, mW1_ref[...], preferred_element_type=_f32) + mb1_ref[...])
    nf_new = (
        nf_ref[...]
        + jnp.dot(t, mW2_ref[...], preferred_element_type=_f32)
        + mb2_ref[...]
    )
    o_ref[...] = nf_new
    h1_ref[...] = jnp.dot(nf_new, lW_ref[...], preferred_element_type=_f32) + lb_ref[...]


_update = pl.pallas_call(
    _update_body,
    grid=(N // NBLK,),
    in_specs=[
        pl.BlockSpec((NC, NBLK, H), lambda i: (0, i, 0)),
        pl.BlockSpec((NBLK, H), lambda i: (i, 0)),
        pl.BlockSpec((H, H), lambda i: (0, 0)),
        pl.BlockSpec((1, H), lambda i: (0, 0)),
        pl.BlockSpec((H, H), lambda i: (0, 0)),
        pl.BlockSpec((1, H), lambda i: (0, 0)),
        pl.BlockSpec((H, H), lambda i: (0, 0)),
        pl.BlockSpec((1, H), lambda i: (0, 0)),
    ],
    out_specs=[pl.BlockSpec((NBLK, H), lambda i: (i, 0)) for _ in range(2)],
    out_shape=[jax.ShapeDtypeStruct((N, H), _f32) for _ in range(2)],
)


def _final_body(nf_ref, batch_ref, Wp1_ref, bp1_ref, Wp2_ref, bp2_ref, o_ref):
    i = pl.program_id(0)
    t = _ssp(jnp.dot(nf_ref[...], Wp1_ref[...], preferred_element_type=_f32) + bp1_ref[...])
    site = jnp.dot(t, Wp2_ref[...], preferred_element_type=_f32) + bp2_ref[...]
    g = lax.broadcasted_iota(jnp.int32, (NG, NBLK), 0)
    mask = (batch_ref[0] == g).astype(_f32)
    contrib = jnp.dot(mask, site, preferred_element_type=_f32, precision=lax.Precision.HIGHEST)

    @pl.when(i == 0)
    def _():
        o_ref[...] = jnp.zeros_like(o_ref)

    o_ref[...] += contrib


_final = pl.pallas_call(
    _final_body,
    grid=(N // NBLK,),
    in_specs=[
        pl.BlockSpec((NBLK, H), lambda i: (i, 0)),
        pl.BlockSpec((1, 1, NBLK), lambda i: (i, 0, 0)),
        pl.BlockSpec((H, H // 2), lambda i: (0, 0)),
        pl.BlockSpec((1, H // 2), lambda i: (0, 0)),
        pl.BlockSpec((H // 2, 1), lambda i: (0, 0)),
        pl.BlockSpec((1, 1), lambda i: (0, 0)),
    ],
    out_specs=pl.BlockSpec((NG, 1), lambda i: (0, 0)),
    out_shape=jax.ShapeDtypeStruct((NG, 1), _f32),
)


def _conv_body(h1_hbm, filt_hbm, src_hbm, dst_hbm, zeros_hbm, out_hbm,
               idx_src_all, idx_dst, rows, filt_v, agg_sh,
               gsem, fsem, dsem, ssem):
    c = lax.axis_index("c")
    s = lax.axis_index("s")
    wid = s * NC + c
    base = pl.multiple_of(wid * EPW, 8)
    pltpu.async_copy(src_hbm.at[pl.ds(base, EPW)], idx_src_all, gsem[0]).wait()
    rb = pl.multiple_of(s * RPB, 8)
    pltpu.sync_copy(zeros_hbm.at[pl.ds(rb, RPB)], agg_sh.at[pl.ds(rb, RPB)])

    @pl.when(s == NS - 1)
    def _():
        pltpu.sync_copy(zeros_hbm.at[pl.ds(NS * RPB, TAIL)],
                        agg_sh.at[pl.ds(NS * RPB, TAIL)])

    plsc.subcore_barrier()

    def issue(k, p, wait_prev):
        if wait_prev:
            pltpu.make_async_copy(h1_hbm.at[pl.ds(0, C)], rows[p], ssem[p]).wait()
        eb = pl.multiple_of(base + k * C, 8)
        pltpu.async_copy(dst_hbm.at[pl.ds(eb, C)], idx_dst[p], dsem[p])
        pltpu.async_copy(h1_hbm.at[idx_src_all.at[pl.ds(pl.multiple_of(k * C, 8), C)]],
                         rows[p], gsem[p])
        pltpu.async_copy(filt_hbm.at[pl.ds(eb, C)], filt_v[p], fsem[p])

    def process(p):
        pltpu.make_async_copy(h1_hbm.at[pl.ds(0, C)], rows[p], gsem[p]).wait()
        pltpu.make_async_copy(filt_hbm.at[pl.ds(0, C)], filt_v[p], fsem[p]).wait()
        pltpu.make_async_copy(dst_hbm.at[pl.ds(0, C)], idx_dst[p], dsem[p]).wait()

        @plsc.parallel_loop(0, C, step=1, unroll=4)
        def _(e):
            for j in range(H // 16):
                sl = pl.ds(j * 16, 16)
                rows[p][e, sl] = rows[p][e, sl] * filt_v[p][e, sl]

        pltpu.async_copy(rows[p], agg_sh.at[idx_dst[p]], ssem[p], add=True)

    issue(0, 0, False)
    issue(1, 1, False)
    process(0)
    issue(2, 2, False)

    def triple(k3, carry):
        a = 1 + 3 * k3
        for j in range(3):
            k = a + j

            @pl.when(k + 2 < NCHUNK)
            def _(k=k, j=j):
                issue(k + 2, j, True)

            process((1 + j) % 3)
        return carry

    lax.fori_loop(0, (NCHUNK - 1) // 3, triple, 0)
    for p in range(3):
        pltpu.make_async_copy(h1_hbm.at[pl.ds(0, C)], rows[p], ssem[p]).wait()
    plsc.subcore_barrier()
    pltpu.sync_copy(agg_sh.at[pl.ds(rb, RPB)], out_hbm.at[c, pl.ds(rb, RPB)])

    @pl.when(s == NS - 1)
    def _():
        pltpu.sync_copy(agg_sh.at[pl.ds(NS * RPB, TAIL)],
                        out_hbm.at[c, pl.ds(NS * RPB, TAIL)])


_conv = pl.kernel(
    _conv_body,
    out_type=jax.ShapeDtypeStruct((NC, N, H), _f32),
    mesh=plsc.VectorSubcoreMesh(
        core_axis_name="c", subcore_axis_name="s", num_cores=NC, num_subcores=NS
    ),
    scratch_types=[
        pltpu.VMEM((EPW,), jnp.int32),
        [pltpu.VMEM((C,), jnp.int32) for _ in range(3)],
        [pltpu.VMEM((C, H), _f32) for _ in range(3)],
        [pltpu.VMEM((C, H), _f32) for _ in range(3)],
        pltpu.VMEM_SHARED((N, H), _f32),
        [pltpu.SemaphoreType.DMA for _ in range(3)],
        [pltpu.SemaphoreType.DMA for _ in range(3)],
        [pltpu.SemaphoreType.DMA for _ in range(3)],
        [pltpu.SemaphoreType.DMA for _ in range(3)],
    ],
)


def kernel(x, edge_index, edge_attr, batch, emb, lin_W, lin_b,
           fW1, fb1, fW2, fb2, mW1, mb1, mW2, mb2, Wp1, bp1, Wp2, bp2):
    src = edge_index[0]
    dst = edge_index[1]
    x2 = x.reshape(N, 1).astype(jnp.int32)
    d2 = edge_attr.reshape(E, 1)
    batch2 = batch.reshape(N // NBLK, 1, NBLK).astype(jnp.int32)
    zeros = jnp.zeros((N, H), _f32)

    nf = _embed(x2, emb)
    h1 = _linear(nf, lin_W[0], lin_b[0].reshape(1, H))
    for b in range(NB):
        filt_b = _filt(d2, fW1[b], fb1[b].reshape(1, H), fW2[b], fb2[b].reshape(1, H))
        aggp = _conv(h1, filt_b, src, dst, zeros)
        nxt = (b + 1) % NB
        nf, h1 = _update(aggp, nf, mW1[b], mb1[b].reshape(1, H),
                         mW2[b], mb2[b].reshape(1, H),
                         lin_W[nxt], lin_b[nxt].reshape(1, H))
    out2 = _final(nf, batch2, Wp1, bp1.reshape(1, H // 2), Wp2, bp2.reshape(1, 1))
    return out2.reshape(NG)


# speedup vs baseline: 5.0544x; 1.0137x over previous
import jax
import jax.numpy as jnp
import numpy as np
from jax import lax
from jax.experimental import pallas as pl
from jax.experimental.pallas import tpu as pltpu
from jax.experimental.pallas import tpu_sc as plsc

H = 128
G = 50
NB = 3
N = 10000
E = 320000
NG = 512
LOG2 = float(np.log(2.0))
STEP = 30.0 / 49.0
COEFF = -0.5 / STEP ** 2

NBLK = 1000
EBLK = 3200
NC, NS = 2, 16
NW = NC * NS
EPW = E // NW
C = 40
NCHUNK = EPW // C
RPB = 624
TAIL = N - NS * RPB

_f32 = jnp.float32


def _ssp(v):
    return jnp.maximum(v, 0.0) + jnp.log(1.0 + jnp.exp(-jnp.abs(v))) - LOG2


def _embed_body(x_ref, emb_ref, lW_ref, lb_ref, o_ref, h1_ref):
    xb = x_ref[...]
    oh = (xb == lax.broadcasted_iota(jnp.int32, (NBLK, 10), 1)).astype(_f32)
    nf = jnp.dot(oh, emb_ref[...], preferred_element_type=_f32, precision=lax.Precision.HIGHEST)
    o_ref[...] = nf
    h1_ref[...] = jnp.dot(nf, lW_ref[...], preferred_element_type=_f32) + lb_ref[...]


_embed = pl.pallas_call(
    _embed_body,
    grid=(N // NBLK,),
    in_specs=[
        pl.BlockSpec((NBLK, 1), lambda i: (i, 0)),
        pl.BlockSpec((10, H), lambda i: (0, 0)),
        pl.BlockSpec((H, H), lambda i: (0, 0)),
        pl.BlockSpec((1, H), lambda i: (0, 0)),
    ],
    out_specs=[pl.BlockSpec((NBLK, H), lambda i: (i, 0)) for _ in range(2)],
    out_shape=[jax.ShapeDtypeStruct((N, H), _f32) for _ in range(2)],
)


def _filt_body(d_ref, fW1_ref, fb1_ref, fW2_ref, fb2_ref, o_ref):
    d = d_ref[...]
    offs = lax.broadcasted_iota(jnp.int32, (1, G), 1).astype(_f32) * STEP
    ea = jnp.exp(COEFF * (d - offs) ** 2)
    t = _ssp(jnp.dot(ea, fW1_ref[...], preferred_element_type=_f32) + fb1_ref[...])
    o_ref[...] = jnp.dot(t, fW2_ref[...], preferred_element_type=_f32) + fb2_ref[...]


_filt = pl.pallas_call(
    _filt_body,
    grid=(E // EBLK,),
    in_specs=[
        pl.BlockSpec((EBLK, 1), lambda i: (i, 0)),
        pl.BlockSpec((G, H), lambda i: (0, 0)),
        pl.BlockSpec((1, H), lambda i: (0, 0)),
        pl.BlockSpec((H, H), lambda i: (0, 0)),
        pl.BlockSpec((1, H), lambda i: (0, 0)),
    ],
    out_specs=pl.BlockSpec((EBLK, H), lambda i: (i, 0)),
    out_shape=jax.ShapeDtypeStruct((E, H), _f32),
)


def _update_body(aggp_ref, nf_ref, mW1_ref, mb1_ref, mW2_ref, mb2_ref,
                 lW_ref, lb_ref, o_ref, h1_ref):
    agg = aggp_ref[0] + aggp_ref[1]
    t = _ssp(jnp.dot(agg, mW1_ref[...], preferred_element_type=_f32) + mb1_ref[...])
    nf_new = (
        nf_ref[...]
        + jnp.dot(t, mW2_ref[...], preferred_element_type=_f32)
        + mb2_ref[...]
    )
    o_ref[...] = nf_new
    h1_ref[...] = jnp.dot(nf_new, lW_ref[...], preferred_element_type=_f32) + lb_ref[...]


_update = pl.pallas_call(
    _update_body,
    grid=(N // NBLK,),
    in_specs=[
        pl.BlockSpec((NC, NBLK, H), lambda i: (0, i, 0)),
        pl.BlockSpec((NBLK, H), lambda i: (i, 0)),
        pl.BlockSpec((H, H), lambda i: (0, 0)),
        pl.BlockSpec((1, H), lambda i: (0, 0)),
        pl.BlockSpec((H, H), lambda i: (0, 0)),
        pl.BlockSpec((1, H), lambda i: (0, 0)),
        pl.BlockSpec((H, H), lambda i: (0, 0)),
        pl.BlockSpec((1, H), lambda i: (0, 0)),
    ],
    out_specs=[pl.BlockSpec((NBLK, H), lambda i: (i, 0)) for _ in range(2)],
    out_shape=[jax.ShapeDtypeStruct((N, H), _f32) for _ in range(2)],
)


def _update_final_body(aggp_ref, nf_ref, mW1_ref, mb1_ref, mW2_ref, mb2_ref,
                       batch_ref, Wp1_ref, bp1_ref, Wp2_ref, bp2_ref, o_ref):
    i = pl.program_id(0)
    agg = aggp_ref[0] + aggp_ref[1]
    t = _ssp(jnp.dot(agg, mW1_ref[...], preferred_element_type=_f32) + mb1_ref[...])
    nf_new = (
        nf_ref[...]
        + jnp.dot(t, mW2_ref[...], preferred_element_type=_f32)
        + mb2_ref[...]
    )
    t2 = _ssp(jnp.dot(nf_new, Wp1_ref[...], preferred_element_type=_f32) + bp1_ref[...])
    site = jnp.dot(t2, Wp2_ref[...], preferred_element_type=_f32) + bp2_ref[...]
    g = lax.broadcasted_iota(jnp.int32, (NG, NBLK), 0)
    mask = (batch_ref[0] == g).astype(_f32)
    contrib = jnp.dot(mask, site, preferred_element_type=_f32, precision=lax.Precision.HIGHEST)

    @pl.when(i == 0)
    def _():
        o_ref[...] = jnp.zeros_like(o_ref)

    o_ref[...] += contrib


_update_final = pl.pallas_call(
    _update_final_body,
    grid=(N // NBLK,),
    in_specs=[
        pl.BlockSpec((NC, NBLK, H), lambda i: (0, i, 0)),
        pl.BlockSpec((NBLK, H), lambda i: (i, 0)),
        pl.BlockSpec((H, H), lambda i: (0, 0)),
        pl.BlockSpec((1, H), lambda i: (0, 0)),
        pl.BlockSpec((H, H), lambda i: (0, 0)),
        pl.BlockSpec((1, H), lambda i: (0, 0)),
        pl.BlockSpec((1, 1, NBLK), lambda i: (i, 0, 0)),
        pl.BlockSpec((H, H // 2), lambda i: (0, 0)),
        pl.BlockSpec((1, H // 2), lambda i: (0, 0)),
        pl.BlockSpec((H // 2, 1), lambda i: (0, 0)),
        pl.BlockSpec((1, 1), lambda i: (0, 0)),
    ],
    out_specs=pl.BlockSpec((NG, 1), lambda i: (0, 0)),
    out_shape=jax.ShapeDtypeStruct((NG, 1), _f32),
)


def _conv_body(h1_hbm, filt_hbm, src_hbm, dst_hbm, zeros_hbm, out_hbm,
               idx_src_all, idx_dst, rows, filt_v, agg_sh,
               gsem, fsem, dsem, ssem):
    c = lax.axis_index("c")
    s = lax.axis_index("s")
    wid = s * NC + c
    base = pl.multiple_of(wid * EPW, 8)
    pltpu.async_copy(src_hbm.at[pl.ds(base, EPW)], idx_src_all, gsem[0]).wait()
    rb = pl.multiple_of(s * RPB, 8)
    pltpu.sync_copy(zeros_hbm.at[pl.ds(rb, RPB)], agg_sh.at[pl.ds(rb, RPB)])

    @pl.when(s == NS - 1)
    def _():
        pltpu.sync_copy(zeros_hbm.at[pl.ds(NS * RPB, TAIL)],
                        agg_sh.at[pl.ds(NS * RPB, TAIL)])

    plsc.subcore_barrier()

    def issue(k, p, wait_prev):
        if wait_prev:
            pltpu.make_async_copy(h1_hbm.at[pl.ds(0, C)], rows[p], ssem[p]).wait()
        eb = pl.multiple_of(base + k * C, 8)
        pltpu.async_copy(dst_hbm.at[pl.ds(eb, C)], idx_dst[p], dsem[p])
        pltpu.async_copy(h1_hbm.at[idx_src_all.at[pl.ds(pl.multiple_of(k * C, 8), C)]],
                         rows[p], gsem[p])
        pltpu.async_copy(filt_hbm.at[pl.ds(eb, C)], filt_v[p], fsem[p])

    def process(p):
        pltpu.make_async_copy(h1_hbm.at[pl.ds(0, C)], rows[p], gsem[p]).wait()
        pltpu.make_async_copy(filt_hbm.at[pl.ds(0, C)], filt_v[p], fsem[p]).wait()
        pltpu.make_async_copy(dst_hbm.at[pl.ds(0, C)], idx_dst[p], dsem[p]).wait()

        @plsc.parallel_loop(0, C, step=1, unroll=4)
        def _(e):
            for j in range(H // 16):
                sl = pl.ds(j * 16, 16)
                rows[p][e, sl] = rows[p][e, sl] * filt_v[p][e, sl]

        pltpu.async_copy(rows[p], agg_sh.at[idx_dst[p]], ssem[p], add=True)

    issue(0, 0, False)
    issue(1, 1, False)
    process(0)
    issue(2, 2, False)

    def triple(k3, carry):
        a = 1 + 3 * k3
        for j in range(3):
            k = a + j

            @pl.when(k + 2 < NCHUNK)
            def _(k=k, j=j):
                issue(k + 2, j, True)

            process((1 + j) % 3)
        return carry

    lax.fori_loop(0, (NCHUNK - 1) // 3, triple, 0)
    for p in range(3):
        pltpu.make_async_copy(h1_hbm.at[pl.ds(0, C)], rows[p], ssem[p]).wait()
    plsc.subcore_barrier()
    pltpu.sync_copy(agg_sh.at[pl.ds(rb, RPB)], out_hbm.at[c, pl.ds(rb, RPB)])

    @pl.when(s == NS - 1)
    def _():
        pltpu.sync_copy(agg_sh.at[pl.ds(NS * RPB, TAIL)],
                        out_hbm.at[c, pl.ds(NS * RPB, TAIL)])


_conv = pl.kernel(
    _conv_body,
    out_type=jax.ShapeDtypeStruct((NC, N, H), _f32),
    mesh=plsc.VectorSubcoreMesh(
        core_axis_name="c", subcore_axis_name="s", num_cores=NC, num_subcores=NS
    ),
    scratch_types=[
        pltpu.VMEM((EPW,), jnp.int32),
        [pltpu.VMEM((C,), jnp.int32) for _ in range(3)],
        [pltpu.VMEM((C, H), _f32) for _ in range(3)],
        [pltpu.VMEM((C, H), _f32) for _ in range(3)],
        pltpu.VMEM_SHARED((N, H), _f32),
        [pltpu.SemaphoreType.DMA for _ in range(3)],
        [pltpu.SemaphoreType.DMA for _ in range(3)],
        [pltpu.SemaphoreType.DMA for _ in range(3)],
        [pltpu.SemaphoreType.DMA for _ in range(3)],
    ],
)


def kernel(x, edge_index, edge_attr, batch, emb, lin_W, lin_b,
           fW1, fb1, fW2, fb2, mW1, mb1, mW2, mb2, Wp1, bp1, Wp2, bp2):
    src = edge_index[0]
    dst = edge_index[1]
    x2 = x.reshape(N, 1).astype(jnp.int32)
    d2 = edge_attr.reshape(E, 1)
    batch2 = batch.reshape(N // NBLK, 1, NBLK).astype(jnp.int32)
    zeros = jnp.zeros((N, H), _f32)

    nf, h1 = _embed(x2, emb, lin_W[0], lin_b[0].reshape(1, H))
    for b in range(NB - 1):
        filt_b = _filt(d2, fW1[b], fb1[b].reshape(1, H), fW2[b], fb2[b].reshape(1, H))
        aggp = _conv(h1, filt_b, src, dst, zeros)
        nf, h1 = _update(aggp, nf, mW1[b], mb1[b].reshape(1, H),
                         mW2[b], mb2[b].reshape(1, H),
                         lin_W[b + 1], lin_b[b + 1].reshape(1, H))
    b = NB - 1
    filt_b = _filt(d2, fW1[b], fb1[b].reshape(1, H), fW2[b], fb2[b].reshape(1, H))
    aggp = _conv(h1, filt_b, src, dst, zeros)
    out2 = _update_final(aggp, nf, mW1[b], mb1[b].reshape(1, H),
                         mW2[b], mb2[b].reshape(1, H),
                         batch2, Wp1, bp1.reshape(1, H // 2), Wp2, bp2.reshape(1, 1))
    return out2.reshape(NG)


# speedup vs baseline: 5.1596x; 1.0208x over previous
import jax
import jax.numpy as jnp
import numpy as np
from jax import lax
from jax.experimental import pallas as pl
from jax.experimental.pallas import tpu as pltpu
from jax.experimental.pallas import tpu_sc as plsc

H = 128
G = 50
NB = 3
N = 10000
E = 320000
NG = 512
LOG2 = float(np.log(2.0))
STEP = 30.0 / 49.0
COEFF = -0.5 / STEP ** 2

NBLK = 1000
EBLK = 8000
NC, NS = 2, 16
NW = NC * NS
EPW = E // NW
C = 40
NCHUNK = EPW // C
RPB = 624
TAIL = N - NS * RPB

_f32 = jnp.float32


def _ssp(v):
    return jnp.maximum(v, 0.0) + jnp.log(1.0 + jnp.exp(-jnp.abs(v))) - LOG2


def _embed_body(x_ref, emb_ref, lW_ref, lb_ref, o_ref, h1_ref):
    xb = x_ref[...]
    oh = (xb == lax.broadcasted_iota(jnp.int32, (NBLK, 10), 1)).astype(_f32)
    nf = jnp.dot(oh, emb_ref[...], preferred_element_type=_f32, precision=lax.Precision.HIGHEST)
    o_ref[...] = nf
    h1_ref[...] = jnp.dot(nf, lW_ref[...], preferred_element_type=_f32) + lb_ref[...]


_embed = pl.pallas_call(
    _embed_body,
    grid=(N // NBLK,),
    in_specs=[
        pl.BlockSpec((NBLK, 1), lambda i: (i, 0)),
        pl.BlockSpec((10, H), lambda i: (0, 0)),
        pl.BlockSpec((H, H), lambda i: (0, 0)),
        pl.BlockSpec((1, H), lambda i: (0, 0)),
    ],
    out_specs=[pl.BlockSpec((NBLK, H), lambda i: (i, 0)) for _ in range(2)],
    out_shape=[jax.ShapeDtypeStruct((N, H), _f32) for _ in range(2)],
)


def _filt_body(d_ref, fW1_ref, fb1_ref, fW2_ref, fb2_ref, o_ref):
    d = d_ref[...]
    offs = lax.broadcasted_iota(jnp.int32, (1, G), 1).astype(_f32) * STEP
    ea = jnp.exp(COEFF * (d - offs) ** 2)
    t = _ssp(jnp.dot(ea, fW1_ref[...], preferred_element_type=_f32) + fb1_ref[...])
    o_ref[...] = jnp.dot(t, fW2_ref[...], preferred_element_type=_f32) + fb2_ref[...]


_filt = pl.pallas_call(
    _filt_body,
    grid=(E // EBLK,),
    in_specs=[
        pl.BlockSpec((EBLK, 1), lambda i: (i, 0)),
        pl.BlockSpec((G, H), lambda i: (0, 0)),
        pl.BlockSpec((1, H), lambda i: (0, 0)),
        pl.BlockSpec((H, H), lambda i: (0, 0)),
        pl.BlockSpec((1, H), lambda i: (0, 0)),
    ],
    out_specs=pl.BlockSpec((EBLK, H), lambda i: (i, 0)),
    out_shape=jax.ShapeDtypeStruct((E, H), _f32),
)


def _update_body(aggp_ref, nf_ref, mW1_ref, mb1_ref, mW2_ref, mb2_ref,
                 lW_ref, lb_ref, o_ref, h1_ref):
    agg = aggp_ref[0] + aggp_ref[1]
    t = _ssp(jnp.dot(agg, mW1_ref[...], preferred_element_type=_f32) + mb1_ref[...])
    nf_new = (
        nf_ref[...]
        + jnp.dot(t, mW2_ref[...], preferred_element_type=_f32)
        + mb2_ref[...]
    )
    o_ref[...] = nf_new
    h1_ref[...] = jnp.dot(nf_new, lW_ref[...], preferred_element_type=_f32) + lb_ref[...]


_update = pl.pallas_call(
    _update_body,
    grid=(N // NBLK,),
    in_specs=[
        pl.BlockSpec((NC, NBLK, H), lambda i: (0, i, 0)),
        pl.BlockSpec((NBLK, H), lambda i: (i, 0)),
        pl.BlockSpec((H, H), lambda i: (0, 0)),
        pl.BlockSpec((1, H), lambda i: (0, 0)),
        pl.BlockSpec((H, H), lambda i: (0, 0)),
        pl.BlockSpec((1, H), lambda i: (0, 0)),
        pl.BlockSpec((H, H), lambda i: (0, 0)),
        pl.BlockSpec((1, H), lambda i: (0, 0)),
    ],
    out_specs=[pl.BlockSpec((NBLK, H), lambda i: (i, 0)) for _ in range(2)],
    out_shape=[jax.ShapeDtypeStruct((N, H), _f32) for _ in range(2)],
)


def _update_final_body(aggp_ref, nf_ref, mW1_ref, mb1_ref, mW2_ref, mb2_ref,
                       batch_ref, Wp1_ref, bp1_ref, Wp2_ref, bp2_ref, o_ref):
    i = pl.program_id(0)
    agg = aggp_ref[0] + aggp_ref[1]
    t = _ssp(jnp.dot(agg, mW1_ref[...], preferred_element_type=_f32) + mb1_ref[...])
    nf_new = (
        nf_ref[...]
        + jnp.dot(t, mW2_ref[...], preferred_element_type=_f32)
        + mb2_ref[...]
    )
    t2 = _ssp(jnp.dot(nf_new, Wp1_ref[...], preferred_element_type=_f32) + bp1_ref[...])
    site = jnp.dot(t2, Wp2_ref[...], preferred_element_type=_f32) + bp2_ref[...]
    g = lax.broadcasted_iota(jnp.int32, (NG, NBLK), 0)
    mask = (batch_ref[0] == g).astype(_f32)
    contrib = jnp.dot(mask, site, preferred_element_type=_f32, precision=lax.Precision.HIGHEST)

    @pl.when(i == 0)
    def _():
        o_ref[...] = jnp.zeros_like(o_ref)

    o_ref[...] += contrib


_update_final = pl.pallas_call(
    _update_final_body,
    grid=(N // NBLK,),
    in_specs=[
        pl.BlockSpec((NC, NBLK, H), lambda i: (0, i, 0)),
        pl.BlockSpec((NBLK, H), lambda i: (i, 0)),
        pl.BlockSpec((H, H), lambda i: (0, 0)),
        pl.BlockSpec((1, H), lambda i: (0, 0)),
        pl.BlockSpec((H, H), lambda i: (0, 0)),
        pl.BlockSpec((1, H), lambda i: (0, 0)),
        pl.BlockSpec((1, 1, NBLK), lambda i: (i, 0, 0)),
        pl.BlockSpec((H, H // 2), lambda i: (0, 0)),
        pl.BlockSpec((1, H // 2), lambda i: (0, 0)),
        pl.BlockSpec((H // 2, 1), lambda i: (0, 0)),
        pl.BlockSpec((1, 1), lambda i: (0, 0)),
    ],
    out_specs=pl.BlockSpec((NG, 1), lambda i: (0, 0)),
    out_shape=jax.ShapeDtypeStruct((NG, 1), _f32),
)


def _conv_body(h1_hbm, filt_hbm, src_hbm, dst_hbm, out_hbm,
               idx_src_all, idx_dst, rows, filt_v, agg_sh,
               gsem, fsem, dsem, ssem):
    c = lax.axis_index("c")
    s = lax.axis_index("s")
    wid = s * NC + c
    base = pl.multiple_of(wid * EPW, 8)
    pltpu.async_copy(src_hbm.at[pl.ds(base, EPW)], idx_src_all, gsem[0]).wait()
    @plsc.parallel_loop(0, C, step=1, unroll=4)
    def _(e):
        for j in range(H // 16):
            rows[0][e, pl.ds(j * 16, 16)] = jnp.zeros((16,), _f32)

    rb = pl.multiple_of(s * RPB, 8)

    def zrow(i, carry):
        pltpu.sync_copy(rows[0].at[pl.ds(0, 24)],
                        agg_sh.at[pl.ds(pl.multiple_of(rb + i * 24, 8), 24)])
        return carry

    lax.fori_loop(0, RPB // 24, zrow, 0)

    @pl.when(s == NS - 1)
    def _():
        pltpu.sync_copy(rows[0].at[pl.ds(0, TAIL)],
                        agg_sh.at[pl.ds(NS * RPB, TAIL)])

    plsc.subcore_barrier()

    def issue(k, p, wait_prev):
        if wait_prev:
            pltpu.make_async_copy(h1_hbm.at[pl.ds(0, C)], rows[p], ssem[p]).wait()
        eb = pl.multiple_of(base + k * C, 8)
        pltpu.async_copy(dst_hbm.at[pl.ds(eb, C)], idx_dst[p], dsem[p])
        pltpu.async_copy(h1_hbm.at[idx_src_all.at[pl.ds(pl.multiple_of(k * C, 8), C)]],
                         rows[p], gsem[p])
        pltpu.async_copy(filt_hbm.at[pl.ds(eb, C)], filt_v[p], fsem[p])

    def process(p):
        pltpu.make_async_copy(h1_hbm.at[pl.ds(0, C)], rows[p], gsem[p]).wait()
        pltpu.make_async_copy(filt_hbm.at[pl.ds(0, C)], filt_v[p], fsem[p]).wait()
        pltpu.make_async_copy(dst_hbm.at[pl.ds(0, C)], idx_dst[p], dsem[p]).wait()

        @plsc.parallel_loop(0, C, step=1, unroll=4)
        def _(e):
            for j in range(H // 16):
                sl = pl.ds(j * 16, 16)
                rows[p][e, sl] = rows[p][e, sl] * filt_v[p][e, sl]

        pltpu.async_copy(rows[p], agg_sh.at[idx_dst[p]], ssem[p], add=True)

    issue(0, 0, False)
    issue(1, 1, False)
    process(0)
    issue(2, 2, False)

    def triple(k3, carry):
        a = 1 + 3 * k3
        for j in range(3):
            k = a + j

            @pl.when(k + 2 < NCHUNK)
            def _(k=k, j=j):
                issue(k + 2, j, True)

            process((1 + j) % 3)
        return carry

    lax.fori_loop(0, (NCHUNK - 1) // 3, triple, 0)
    for p in range(3):
        pltpu.make_async_copy(h1_hbm.at[pl.ds(0, C)], rows[p], ssem[p]).wait()
    plsc.subcore_barrier()
    pltpu.sync_copy(agg_sh.at[pl.ds(rb, RPB)], out_hbm.at[c, pl.ds(rb, RPB)])

    @pl.when(s == NS - 1)
    def _():
        pltpu.sync_copy(agg_sh.at[pl.ds(NS * RPB, TAIL)],
                        out_hbm.at[c, pl.ds(NS * RPB, TAIL)])


_conv = pl.kernel(
    _conv_body,
    out_type=jax.ShapeDtypeStruct((NC, N, H), _f32),
    mesh=plsc.VectorSubcoreMesh(
        core_axis_name="c", subcore_axis_name="s", num_cores=NC, num_subcores=NS
    ),
    scratch_types=[
        pltpu.VMEM((EPW,), jnp.int32),
        [pltpu.VMEM((C,), jnp.int32) for _ in range(3)],
        [pltpu.VMEM((C, H), _f32) for _ in range(3)],
        [pltpu.VMEM((C, H), _f32) for _ in range(3)],
        pltpu.VMEM_SHARED((N, H), _f32),
        [pltpu.SemaphoreType.DMA for _ in range(3)],
        [pltpu.SemaphoreType.DMA for _ in range(3)],
        [pltpu.SemaphoreType.DMA for _ in range(3)],
        [pltpu.SemaphoreType.DMA for _ in range(3)],
    ],
)


def kernel(x, edge_index, edge_attr, batch, emb, lin_W, lin_b,
           fW1, fb1, fW2, fb2, mW1, mb1, mW2, mb2, Wp1, bp1, Wp2, bp2):
    src = edge_index[0]
    dst = edge_index[1]
    x2 = x.reshape(N, 1).astype(jnp.int32)
    d2 = edge_attr.reshape(E, 1)
    batch2 = batch.reshape(N // NBLK, 1, NBLK).astype(jnp.int32)

    nf, h1 = _embed(x2, emb, lin_W[0], lin_b[0].reshape(1, H))
    for b in range(NB - 1):
        filt_b = _filt(d2, fW1[b], fb1[b].reshape(1, H), fW2[b], fb2[b].reshape(1, H))
        aggp = _conv(h1, filt_b, src, dst)
        nf, h1 = _update(aggp, nf, mW1[b], mb1[b].reshape(1, H),
                         mW2[b], mb2[b].reshape(1, H),
                         lin_W[b + 1], lin_b[b + 1].reshape(1, H))
    b = NB - 1
    filt_b = _filt(d2, fW1[b], fb1[b].reshape(1, H), fW2[b], fb2[b].reshape(1, H))
    aggp = _conv(h1, filt_b, src, dst)
    out2 = _update_final(aggp, nf, mW1[b], mb1[b].reshape(1, H),
                         mW2[b], mb2[b].reshape(1, H),
                         batch2, Wp1, bp1.reshape(1, H // 2), Wp2, bp2.reshape(1, 1))
    return out2.reshape(NG)
